# Initial kernel scaffold; baseline (speedup 1.0000x reference)
import jax
import jax.numpy as jnp
from jax.experimental import pallas as pl


def kernel(x, edge_index, edge_vals, W1, b1, W2, b2, W3, b3):
    raise NotImplementedError("write your pallas kernel here")


# speedup vs baseline: 3.6311x; 3.6311x over previous
import functools

import jax
import jax.numpy as jnp
from jax import lax
from jax.experimental import pallas as pl
from jax.experimental.pallas import tpu as pltpu
from jax.experimental.pallas import tpu_sc as plsc

N = 10000
E = 320000

NC = 2
NS = 16
EPW = E // (NC * NS)
K = 80
NCH = EPW // K
NP = 10240
RPT = NP // NS
RCH = 128
NRC = RPT // RCH


def _spmm_sc(z, src, dst, ev, hf):
    mesh = plsc.VectorSubcoreMesh(core_axis_name="c", subcore_axis_name="s")

    @functools.partial(
        pl.kernel,
        mesh=mesh,
        out_type=jax.ShapeDtypeStruct((NC * NP, hf), jnp.float32),
        scratch_types=[
            pltpu.VMEM_SHARED((NP, hf), jnp.float32),
            pltpu.VMEM((K,), jnp.int32),
            pltpu.VMEM((K,), jnp.int32),
            pltpu.VMEM((K,), jnp.float32),
            pltpu.VMEM((K, hf), jnp.float32),
            pltpu.VMEM((RCH, hf), jnp.float32),
            pltpu.SemaphoreType.DMA,
        ],
    )
    def k(z_hbm, src_hbm, dst_hbm, ev_hbm, out_hbm,
          acc, sbuf, dbuf, ebuf, rows, zbuf, sem):
        c = lax.axis_index("c")
        s = lax.axis_index("s")
        wid = c * NS + s

        def zrow(r, carry):
            for j in range(hf // 16):
                zbuf[r, pl.ds(16 * j, 16)] = jnp.zeros((16,), jnp.float32)
            return carry
        lax.fori_loop(0, RCH, zrow, 0)

        def zcp(t, carry):
            pltpu.sync_copy(zbuf, acc.at[pl.ds(s * RPT + t * RCH, RCH)])
            return carry
        lax.fori_loop(0, NRC, zcp, 0)
        plsc.subcore_barrier()

        def chunk(g, carry):
            base = pl.multiple_of(wid * EPW + g * K, K)
            pltpu.sync_copy(src_hbm.at[pl.ds(base, K)], sbuf)
            pltpu.sync_copy(dst_hbm.at[pl.ds(base, K)], dbuf)
            pltpu.sync_copy(ev_hbm.at[pl.ds(base, K)], ebuf)
            pltpu.async_copy(z_hbm.at[sbuf], rows, sem).wait()

            def scale(q, carry2):
                evv = ebuf[pl.ds(16 * q, 16)]
                for l in range(16):
                    ev_s = evv[l]
                    r = 16 * q + l
                    for j in range(hf // 16):
                        sl = pl.ds(16 * j, 16)
                        rows[r, sl] = rows[r, sl] * ev_s
                return carry2
            lax.fori_loop(0, K // 16, scale, 0)

            pltpu.sync_copy(rows, acc.at[dbuf], add=True)
            return carry
        lax.fori_loop(0, NCH, chunk, 0)
        plsc.subcore_barrier()

        def cpo(t, carry):
            r0 = s * RPT + t * RCH
            pltpu.sync_copy(acc.at[pl.ds(r0, RCH)], zbuf)
            pltpu.sync_copy(zbuf, out_hbm.at[pl.ds(c * NP + r0, RCH)])
            return carry
        lax.fori_loop(0, NRC, cpo, 0)

    return k(z, src, dst, ev)


BN = 2000


def _mm1(x, w, b):
    hf = w.shape[1]

    def body(x_ref, w_ref, b_ref, o_ref):
        o_ref[...] = jnp.dot(x_ref[...], w_ref[...],
                             preferred_element_type=jnp.float32) + b_ref[...]

    return pl.pallas_call(
        body,
        grid=(N // BN,),
        in_specs=[
            pl.BlockSpec((BN, x.shape[1]), lambda i: (i, 0)),
            pl.BlockSpec(w.shape, lambda i: (0, 0)),
            pl.BlockSpec((1, hf), lambda i: (0, 0)),
        ],
        out_specs=pl.BlockSpec((BN, hf), lambda i: (i, 0)),
        out_shape=jax.ShapeDtypeStruct((N, hf), jnp.float32),
    )(x, w, b.reshape(1, hf))


def _mm2(p0, p1, w, b):
    hf = w.shape[1]

    def body(p0_ref, p1_ref, w_ref, b_ref, o_ref):
        h = jnp.maximum(p0_ref[...] + p1_ref[...], 0.0)
        o_ref[...] = jnp.dot(h, w_ref[...],
                             preferred_element_type=jnp.float32) + b_ref[...]

    return pl.pallas_call(
        body,
        grid=(N // BN,),
        in_specs=[
            pl.BlockSpec((BN, p0.shape[1]), lambda i: (i, 0)),
            pl.BlockSpec((BN, p1.shape[1]), lambda i: (i, 0)),
            pl.BlockSpec(w.shape, lambda i: (0, 0)),
            pl.BlockSpec((1, hf), lambda i: (0, 0)),
        ],
        out_specs=pl.BlockSpec((BN, hf), lambda i: (i, 0)),
        out_shape=jax.ShapeDtypeStruct((N, hf), jnp.float32),
    )(p0, p1, w, b.reshape(1, hf))


def _addp(p0, p1):
    hf = p0.shape[1]

    def body(p0_ref, p1_ref, o_ref):
        o_ref[...] = p0_ref[...] + p1_ref[...]

    return pl.pallas_call(
        body,
        grid=(N // BN,),
        in_specs=[
            pl.BlockSpec((BN, hf), lambda i: (i, 0)),
            pl.BlockSpec((BN, hf), lambda i: (i, 0)),
        ],
        out_specs=pl.BlockSpec((BN, hf), lambda i: (i, 0)),
        out_shape=jax.ShapeDtypeStruct((N, hf), jnp.float32),
    )(p0, p1)


def kernel(x, edge_index, edge_vals, W1, b1, W2, b2, W3, b3):
    src = edge_index[1]
    dst = edge_index[0]

    z = _mm1(x, W1, b1)
    p = _spmm_sc(z, src, dst, edge_vals, W1.shape[1])
    z = _mm2(p[:N], p[NP:NP + N], W2, b2)
    p = _spmm_sc(z, src, dst, edge_vals, W2.shape[1])
    c = W3.shape[1]
    W3p = jnp.pad(W3, ((0, 0), (0, 128 - c)))
    b3p = jnp.pad(b3, (0, 128 - c))
    z = _mm2(p[:N], p[NP:NP + N], W3p, b3p)
    p = _spmm_sc(z, src, dst, edge_vals, 128)
    return _addp(p[:N], p[NP:NP + N])[:, :c]
